# baseline (speedup 1.0000x reference)
import functools

import jax
import jax.numpy as jnp
from jax.experimental import pallas as pl

_COMMIT = 0.25
_TILE = 256


def _rqvae_body(x_ref, w1_ref, b1_ref, w2_ref, b2_ref,
                dw1_ref, db1_ref, dw2_ref, db2_ref,
                cb0_ref, cb1_ref, cb2_ref,
                recon_ref, loss_ref, i0_ref, i1_ref, i2_ref,
                *, n_tiles, inv_count):
    hi = jax.lax.Precision.HIGHEST
    x = x_ref[...]
    h = jnp.maximum(jnp.dot(x, w1_ref[...]) + b1_ref[...], 0.0)
    z = jnp.dot(h, w2_ref[...]) + b2_ref[...]

    residual = z
    qsum = jnp.zeros_like(z)
    loss_acc = jnp.float32(0.0)
    for cb_ref, idx_ref in ((cb0_ref, i0_ref), (cb1_ref, i1_ref), (cb2_ref, i2_ref)):
        cb = cb_ref[...]
        cb_norm = jnp.sum(cb * cb, axis=1)
        d = (jnp.sum(residual * residual, axis=1, keepdims=True)
             - 2.0 * jnp.dot(residual, cb.T)
             + cb_norm[None, :])
        idx = jnp.argmin(d, axis=1).astype(jnp.int32)
        onehot = (jax.lax.broadcasted_iota(jnp.int32, d.shape, 1)
                  == idx[:, None]).astype(jnp.float32)
        zq = jax.lax.dot(onehot, cb, precision=jax.lax.Precision.HIGHEST)
        t = zq - residual
        zq_st = residual + t
        loss_acc = loss_acc + jnp.sum(t * t)
        residual = residual - zq_st
        qsum = qsum + zq_st
        idx_ref[...] = idx[:, None]
    rec = jnp.maximum(jnp.dot(qsum, dw1_ref[...]) + db1_ref[...], 0.0)
    recon_ref[...] = jnp.dot(rec, dw2_ref[...]) + db2_ref[...]

    i = pl.program_id(0)

    @pl.when(i == 0)
    def _init():
        loss_ref[...] = jnp.zeros((1, 1), jnp.float32)

    loss_ref[...] += jnp.reshape(loss_acc, (1, 1))

    @pl.when(i == n_tiles - 1)
    def _finish():
        loss_ref[...] = loss_ref[...] * inv_count


def kernel(x, enc_W1, enc_b1, enc_W2, enc_b2, dec_W1, dec_b1, dec_W2, dec_b2, cb0, cb1, cb2):
    batch, in_dim = x.shape
    emb = enc_W2.shape[1]
    n_tiles = batch // _TILE

    body = functools.partial(
        _rqvae_body, n_tiles=n_tiles,
        inv_count=_COMMIT / (batch * emb))

    full = lambda a: pl.BlockSpec(a.shape, lambda i: (0,) * a.ndim)
    b1 = enc_b1.reshape(1, -1)
    b2 = enc_b2.reshape(1, -1)
    db1 = dec_b1.reshape(1, -1)
    db2 = dec_b2.reshape(1, -1)

    recon, loss, i0, i1, i2 = pl.pallas_call(
        body,
        grid=(n_tiles,),
        in_specs=[
            pl.BlockSpec((_TILE, in_dim), lambda i: (i, 0)),
            full(enc_W1), full(b1), full(enc_W2), full(b2),
            full(dec_W1), full(db1), full(dec_W2), full(db2),
            full(cb0), full(cb1), full(cb2),
        ],
        out_specs=[
            pl.BlockSpec((_TILE, emb), lambda i: (i, 0)),
            pl.BlockSpec((1, 1), lambda i: (0, 0)),
            pl.BlockSpec((_TILE, 1), lambda i: (i, 0)),
            pl.BlockSpec((_TILE, 1), lambda i: (i, 0)),
            pl.BlockSpec((_TILE, 1), lambda i: (i, 0)),
        ],
        out_shape=[
            jax.ShapeDtypeStruct((batch, emb), jnp.float32),
            jax.ShapeDtypeStruct((1, 1), jnp.float32),
            jax.ShapeDtypeStruct((batch, 1), jnp.int32),
            jax.ShapeDtypeStruct((batch, 1), jnp.int32),
            jax.ShapeDtypeStruct((batch, 1), jnp.int32),
        ],
    )(x, enc_W1, b1, enc_W2, b2, dec_W1, db1, dec_W2, db2, cb0, cb1, cb2)

    codes = jnp.concatenate([i0, i1, i2], axis=1)
    return recon, loss[0, 0], codes
